# baseline (speedup 1.0000x reference)
import functools

import jax
import jax.numpy as jnp
from jax import lax
from jax.experimental import pallas as pl
from jax.experimental.pallas import tpu as pltpu
from jax.experimental.pallas import tpu_sc as plsc

B = 16384
D = 32
L = 16
NC = 2
NS = 16
NW = NC * NS
BPW = B // NW
NIDX = 4
ICH = BPW // NIDX
NG = BPW // L
NROW = 1000000
GRID = 1954
QS = GRID * 128


def _tr_body(v0_ref, v1_ref, v2_ref, v3_ref, out_ref):
    for t, ref in enumerate((v0_ref, v1_ref, v2_ref, v3_ref)):
        out_ref[:, t * D:(t + 1) * D] = ref[...].T


_transpose_tc = pl.pallas_call(
    _tr_body,
    grid=(GRID,),
    in_specs=[pl.BlockSpec((D, 128), lambda i, t=t: (0, i + GRID * t))
              for t in range(4)],
    out_specs=pl.BlockSpec((128, 128), lambda i: (i, 0)),
    out_shape=jax.ShapeDtypeStruct((QS, 128), jnp.float32),
)


def _pack(vt):
    return _transpose_tc(vt, vt, vt, vt)


NGC = ICH // L


def _gmf_body(uidx_hbm, iidx_hbm, uoff_hbm, uemb_hbm, iemb_hbm, wtb_hbm,
              out_hbm, uidx_v, iidx_v, uoff_v, ue_v, ie_v, wtb_v, out_v,
              usem, isem):
    wid = lax.axis_index("s") * NC + lax.axis_index("c")
    base = wid * BPW
    pltpu.sync_copy(uidx_hbm.at[wid], uidx_v)
    pltpu.sync_copy(iidx_hbm.at[wid], iidx_v)
    pltpu.sync_copy(uoff_hbm.at[wid], uoff_v)
    pltpu.sync_copy(wtb_hbm, wtb_v)

    lane = lax.iota(jnp.int32, L)
    w0 = wtb_v[pl.ds(0, L)]
    w1 = wtb_v[pl.ds(L, L)]
    bias = wtb_v[pl.ds(2 * L, L)]

    def fire(j, slot):
        return (pltpu.async_copy(uemb_hbm.at[uidx_v.at[j]],
                                 ue_v.at[slot], usem),
                pltpu.async_copy(iemb_hbm.at[iidx_v.at[j]],
                                 ie_v.at[slot], isem))

    inflight = {0: fire(0, 0)}
    for j in range(NIDX):
        if j + 1 < NIDX:
            inflight[j + 1] = fire(j + 1, (j + 1) % 2)
        for c in inflight.pop(j):
            c.wait()
        slot = j % 2

        def group(g, carry):
            acc = bias
            offs_vec = uoff_v[pl.ds(j * ICH + g * L, L)]
            for r in range(L):
                row = g * L + r
                offs = offs_vec[r]
                uo = lax.rem(offs, 4) * 32
                io = lax.div(offs, 4) * 32
                rows = jnp.full((L,), row, dtype=jnp.int32)
                u0 = plsc.load_gather(ue_v.at[slot], [rows, uo + lane])
                u1 = plsc.load_gather(ue_v.at[slot], [rows, uo + lane + L])
                i0 = plsc.load_gather(ie_v.at[slot], [rows, io + lane])
                i1 = plsc.load_gather(ie_v.at[slot], [rows, io + lane + L])
                p = u0 * i0 * w0 + u1 * i1 * w1
                s = jnp.sum(p)
                acc = jnp.where(lane == r, s, acc)
            out_v[pl.ds(j * ICH + g * L, L)] = acc + bias
            return carry

        lax.fori_loop(0, NGC, group, 0)
    pltpu.sync_copy(out_v, out_hbm.at[pl.ds(base, BPW)])


_gmf_sc = functools.partial(
    pl.kernel,
    mesh=plsc.VectorSubcoreMesh(core_axis_name="c", subcore_axis_name="s"),
    out_type=jax.ShapeDtypeStruct((B,), jnp.float32),
    scratch_types=[
        pltpu.VMEM((NIDX, ICH), jnp.int32),
        pltpu.VMEM((NIDX, ICH), jnp.int32),
        pltpu.VMEM((BPW,), jnp.int32),
        pltpu.VMEM((2, ICH, 128), jnp.float32),
        pltpu.VMEM((2, ICH, 128), jnp.float32),
        pltpu.VMEM((D + L,), jnp.float32),
        pltpu.VMEM((BPW,), jnp.float32),
        pltpu.SemaphoreType.DMA,
        pltpu.SemaphoreType.DMA,
    ],
    compiler_params=pltpu.CompilerParams(needs_layout_passes=False,
                                         use_tc_tiling_on_sc=True),
)(_gmf_body)


def kernel(user_idx, item_idx, user_emb, item_emb, head_w, head_b,
           user_bias, item_bias, global_bias):
    del user_bias, item_bias
    ui = user_idx.astype(jnp.int32)
    ii = item_idx.astype(jnp.int32)
    uidx = (ui % QS).reshape(NW, NIDX, ICH)
    iidx = (ii % QS).reshape(NW, NIDX, ICH)
    uoff = (ui // QS + 4 * (ii // QS)).reshape(NW, BPW)
    wtb = jnp.concatenate(
        [head_w.reshape(D),
         jnp.broadcast_to((head_b + global_bias).reshape(1), (L,))])
    up = _pack(user_emb.T)
    ip = _pack(item_emb.T)
    return _gmf_sc(uidx, iidx, uoff, up, ip, wtb)


# speedup vs baseline: 4.5255x; 4.5255x over previous
import functools

import jax
import jax.numpy as jnp
from jax import lax
from jax.experimental import pallas as pl
from jax.experimental.pallas import tpu as pltpu
from jax.experimental.pallas import tpu_sc as plsc

B = 16384
D = 32
L = 16
NC = 2
NS = 16
NW = NC * NS
BPW = B // NW
NIDX = 4
ICH = BPW // NIDX
NG = BPW // L
NROW = 1000000
GRID = 49
CB = 5120
QS = GRID * CB


def _tr_body(v0_ref, v1_ref, v2_ref, v3_ref, out_ref):
    for t, ref in enumerate((v0_ref, v1_ref, v2_ref, v3_ref)):
        out_ref[:, t * D:(t + 1) * D] = ref[...].T


_transpose_tc = pl.pallas_call(
    _tr_body,
    grid=(GRID,),
    in_specs=[pl.BlockSpec((D, CB), lambda i, t=t: (0, i + GRID * t))
              for t in range(4)],
    out_specs=pl.BlockSpec((CB, 128), lambda i: (i, 0)),
    out_shape=jax.ShapeDtypeStruct((QS, 128), jnp.float32),
)


def _pack(vt):
    return _transpose_tc(vt, vt, vt, vt)


NGC = ICH // L


def _gmf_body(uidx_hbm, iidx_hbm, uoff_hbm, uemb_hbm, iemb_hbm, wtb_hbm,
              out_hbm, uidx_v, iidx_v, uoff_v, ue_v, ie_v, wtb_v, out_v,
              usem, isem):
    wid = lax.axis_index("s") * NC + lax.axis_index("c")
    base = wid * BPW
    pltpu.sync_copy(uidx_hbm.at[wid], uidx_v)
    pltpu.sync_copy(iidx_hbm.at[wid], iidx_v)
    pltpu.sync_copy(uoff_hbm.at[wid], uoff_v)
    pltpu.sync_copy(wtb_hbm, wtb_v)

    lane = lax.iota(jnp.int32, L)
    w0 = wtb_v[pl.ds(0, L)]
    w1 = wtb_v[pl.ds(L, L)]
    bias = wtb_v[pl.ds(2 * L, L)]

    def fire(j, slot):
        return (pltpu.async_copy(uemb_hbm.at[uidx_v.at[j]],
                                 ue_v.at[slot], usem),
                pltpu.async_copy(iemb_hbm.at[iidx_v.at[j]],
                                 ie_v.at[slot], isem))

    inflight = {0: fire(0, 0)}
    for j in range(NIDX):
        if j + 1 < NIDX:
            inflight[j + 1] = fire(j + 1, (j + 1) % 2)
        for c in inflight.pop(j):
            c.wait()
        slot = j % 2

        def group(g, carry):
            acc = bias
            offs_vec = uoff_v[pl.ds(j * ICH + g * L, L)]
            for r in range(L):
                row = g * L + r
                offs = offs_vec[r]
                uo = lax.rem(offs, 4) * 32
                io = lax.div(offs, 4) * 32
                rows = jnp.full((L,), row, dtype=jnp.int32)
                u0 = plsc.load_gather(ue_v.at[slot], [rows, uo + lane])
                u1 = plsc.load_gather(ue_v.at[slot], [rows, uo + lane + L])
                i0 = plsc.load_gather(ie_v.at[slot], [rows, io + lane])
                i1 = plsc.load_gather(ie_v.at[slot], [rows, io + lane + L])
                p = u0 * i0 * w0 + u1 * i1 * w1
                s = jnp.sum(p)
                acc = jnp.where(lane == r, s, acc)
            out_v[pl.ds(j * ICH + g * L, L)] = acc + bias
            return carry

        lax.fori_loop(0, NGC, group, 0)
    pltpu.sync_copy(out_v, out_hbm.at[pl.ds(base, BPW)])


_gmf_sc = functools.partial(
    pl.kernel,
    mesh=plsc.VectorSubcoreMesh(core_axis_name="c", subcore_axis_name="s"),
    out_type=jax.ShapeDtypeStruct((B,), jnp.float32),
    scratch_types=[
        pltpu.VMEM((NIDX, ICH), jnp.int32),
        pltpu.VMEM((NIDX, ICH), jnp.int32),
        pltpu.VMEM((BPW,), jnp.int32),
        pltpu.VMEM((2, ICH, 128), jnp.float32),
        pltpu.VMEM((2, ICH, 128), jnp.float32),
        pltpu.VMEM((D + L,), jnp.float32),
        pltpu.VMEM((BPW,), jnp.float32),
        pltpu.SemaphoreType.DMA,
        pltpu.SemaphoreType.DMA,
    ],
    compiler_params=pltpu.CompilerParams(needs_layout_passes=False,
                                         use_tc_tiling_on_sc=True),
)(_gmf_body)


def kernel(user_idx, item_idx, user_emb, item_emb, head_w, head_b,
           user_bias, item_bias, global_bias):
    del user_bias, item_bias
    ui = user_idx.astype(jnp.int32)
    ii = item_idx.astype(jnp.int32)
    uidx = (ui % QS).reshape(NW, NIDX, ICH)
    iidx = (ii % QS).reshape(NW, NIDX, ICH)
    uoff = (ui // QS + 4 * (ii // QS)).reshape(NW, BPW)
    wtb = jnp.concatenate(
        [head_w.reshape(D),
         jnp.broadcast_to((head_b + global_bias).reshape(1), (L,))])
    up = _pack(user_emb.T)
    ip = _pack(item_emb.T)
    return _gmf_sc(uidx, iidx, uoff, up, ip, wtb)
